# baseline (speedup 1.0000x reference)
import dataclasses
import functools

import numpy as np
import jax
import jax.numpy as jnp
from jax import lax
from jax.experimental import pallas as pl
from jax.experimental.pallas import tpu as pltpu
from jax.experimental.pallas import tpu_sc as plsc

N = 10000
E = 320000
D = 128
NUM_E = E + N

_NC = 2
_NS = 16
_L = 16
_NW = _NC * _NS
_C = 128
_NCH = 80
_EPW = _NCH * _C
_EPAD = _NW * _EPW
_HNP = 10112
_WSTEP = 624
_WR = 640


def _threefry2x32(k1, k2, x0, x1):
    rot_a = (13, 15, 26, 6)
    rot_b = (17, 29, 16, 24)
    u32 = np.uint32
    ks = (k1, k2, u32(k1 ^ k2 ^ u32(0x1BD11BDA)))
    x0 = (x0 + ks[0]).astype(u32)
    x1 = (x1 + ks[1]).astype(u32)

    def rounds(x0, x1, rots):
        for r in rots:
            x0 = (x0 + x1).astype(u32)
            x1 = ((x1 << u32(r)) | (x1 >> u32(32 - r))).astype(u32)
            x1 = x0 ^ x1
        return x0, x1

    for i, rots in enumerate((rot_a, rot_b, rot_a, rot_b, rot_a)):
        x0, x1 = rounds(x0, x1, rots)
        x0 = (x0 + ks[(i + 1) % 3]).astype(u32)
        x1 = (x1 + ks[(i + 2) % 3] + u32(i + 1)).astype(u32)
    return x0, x1


def _np_random_bits(key, n):
    b1, b2 = _threefry2x32(key[0], key[1],
                           np.zeros(n, np.uint32), np.arange(n, dtype=np.uint32))
    return b1 ^ b2


def _drop_mask() -> np.ndarray:
    u32 = np.uint32
    keep_prob = 1.0 - 0.3
    drop_size = int((1.0 - keep_prob) * NUM_E)
    key = (u32(0), u32(42))
    s1, s2 = _threefry2x32(key[0], key[1],
                           np.zeros(2, np.uint32), np.arange(2, dtype=np.uint32))
    k1 = (s1[0], s2[0])
    k2 = (s1[1], s2[1])
    higher = _np_random_bits(k1, drop_size)
    lower = _np_random_bits(k2, drop_size)
    span = u32(NUM_E)
    multiplier = u32((int(2 ** 16 % int(span)) ** 2 % 2 ** 32) % int(span))
    offset = ((higher % span) * multiplier + lower % span).astype(u32) % span
    drop_idx = offset.astype(np.int32)
    m = np.ones((NUM_E,), np.float32)
    m[drop_idx] = 0.0
    return m


_MASK = _drop_mask()
_MASK_E = _MASK[:E]
_S_LOOP = _MASK[E:]


def _deg_body(src_hbm, dst_hbm, out_hbm, idx_v, hs_v, hd_v):
    cid = lax.axis_index("c")
    sid = lax.axis_index("s")
    wid = sid * _NC + cid

    zeros = jnp.zeros((_L,), jnp.float32)

    @pl.loop(0, _HNP // _L)
    def _zero(r):
        hs_v[pl.ds(r * _L, _L)] = zeros
        hd_v[pl.ds(r * _L, _L)] = zeros

    ones = jnp.ones((_L,), jnp.float32)
    for t in range(2):
        src = (src_hbm, dst_hbm)[t]
        hist = (hs_v, hd_v)[t]
        pltpu.sync_copy(src.at[pl.ds(wid * _EPW, _EPW)], idx_v)

        @pl.loop(0, _EPW // _L)
        def _acc(g):
            iv = idx_v[pl.ds(g * _L, _L)]
            plsc.addupdate_scatter(hist, [iv], ones)

    pltpu.sync_copy(hs_v, out_hbm.at[wid, 0])
    pltpu.sync_copy(hd_v, out_hbm.at[wid, 1])


def _sc_compiler_params():
    cp = pltpu.CompilerParams()
    if "needs_layout_passes" in pltpu.CompilerParams.__dataclass_fields__:
        cp = dataclasses.replace(cp, needs_layout_passes=False)
    return cp


@functools.cache
def _deg_kernel():
    mesh = plsc.VectorSubcoreMesh(core_axis_name="c", subcore_axis_name="s")
    return pl.kernel(
        _deg_body,
        out_type=jax.ShapeDtypeStruct((_NW, 2, _HNP), jnp.float32),
        mesh=mesh,
        scratch_types=[
            pltpu.VMEM((_EPW,), jnp.int32),
            pltpu.VMEM((_HNP,), jnp.float32),
            pltpu.VMEM((_HNP,), jnp.float32),
        ],
        compiler_params=_sc_compiler_params(),
    )


_BLK = 16
_NCH0 = 16
_NCH1 = 2 * _NCH - _NCH0


def _scale_rows(rows, cfv, bc):
    @pl.loop(0, _C // _L)
    def _group(g):
        cv = cfv[bc, pl.ds(g * _L, _L)]
        for i in range(_L):
            wb = jnp.broadcast_to(cv[i], (_L,))
            r = g * _L + i
            for j in range(D // _L):
                fs = pl.ds(j * _L, _L)
                rows[r, fs] = rows[r, fs] * wb


def _conv_body(h_hbm, z_hbm, src_hbm, dst_hbm, cf_hbm, out_hbm,
               sidx, didx, cfv, rows0, rows1, sem0, sem1, acc_sh):
    cid = lax.axis_index("c")
    sid = lax.axis_index("s")

    row0 = sid * _WSTEP
    pltpu.sync_copy(z_hbm.at[pl.ds(row0, _WR), :],
                    acc_sh.at[pl.ds(row0, _WR), :])
    plsc.subcore_barrier()

    cbase = jnp.where(cid == 0, sid * _NCH0, _NS * _NCH0 + sid * _NCH1)
    nblk = jnp.where(cid == 0, _NCH0 // _BLK, _NCH1 // _BLK)

    @pl.loop(0, nblk)
    def _block(b):
        bb = cbase + b * _BLK
        pltpu.sync_copy(src_hbm.at[pl.ds(bb, _BLK), :], sidx)
        pltpu.sync_copy(dst_hbm.at[pl.ds(bb, _BLK), :], didx)
        pltpu.sync_copy(cf_hbm.at[pl.ds(bb, _BLK), :], cfv)

        pltpu.async_copy(h_hbm.at[sidx.at[0]], rows0, sem0)

        @pl.loop(0, _BLK // 2)
        def _pair(p):
            c0 = 2 * p
            c1 = 2 * p + 1
            pltpu.make_async_copy(h_hbm.at[sidx.at[c0]], rows0, sem0).wait()
            pltpu.async_copy(h_hbm.at[sidx.at[c1]], rows1, sem1)
            _scale_rows(rows0, cfv, c0)
            pltpu.sync_copy(rows0, acc_sh.at[didx.at[c0]], add=True)
            pltpu.make_async_copy(h_hbm.at[sidx.at[c1]], rows1, sem1).wait()

            @pl.when(p < _BLK // 2 - 1)
            def _prefetch():
                pltpu.async_copy(h_hbm.at[sidx.at[c1 + 1]], rows0, sem0)

            _scale_rows(rows1, cfv, c1)
            pltpu.sync_copy(rows1, acc_sh.at[didx.at[c1]], add=True)

    plsc.subcore_barrier()
    pltpu.sync_copy(acc_sh.at[pl.ds(row0, _WR), :],
                    out_hbm.at[cid, pl.ds(row0, _WR), :])


@functools.cache
def _conv_kernel():
    mesh = plsc.VectorSubcoreMesh(core_axis_name="c", subcore_axis_name="s")
    return pl.kernel(
        _conv_body,
        out_type=jax.ShapeDtypeStruct((_NC, N, D), jnp.float32),
        mesh=mesh,
        scratch_types=[
            pltpu.VMEM((_BLK, _C), jnp.int32),
            pltpu.VMEM((_BLK, _C), jnp.int32),
            pltpu.VMEM((_BLK, _C), jnp.float32),
            pltpu.VMEM((_C, D), jnp.float32),
            pltpu.VMEM((_C, D), jnp.float32),
            pltpu.SemaphoreType.DMA,
            pltpu.SemaphoreType.DMA,
            pltpu.VMEM_SHARED((N, D), jnp.float32),
        ],
        compiler_params=_sc_compiler_params(),
    )


def _coef_body(src_hbm, dst_hbm, we_hbm, ns_hbm, nd_hbm, cf_hbm,
               sv, dv, wv, cv, ns_v, nd_v):
    cid = lax.axis_index("c")
    sid = lax.axis_index("s")
    wid = sid * _NC + cid
    base = wid * _EPW

    pltpu.sync_copy(ns_hbm, ns_v)
    pltpu.sync_copy(nd_hbm, nd_v)
    pltpu.sync_copy(src_hbm.at[pl.ds(base, _EPW)], sv)
    pltpu.sync_copy(dst_hbm.at[pl.ds(base, _EPW)], dv)
    pltpu.sync_copy(we_hbm.at[pl.ds(base, _EPW)], wv)

    @pl.loop(0, _EPW // _L)
    def _group(g):
        sl = pl.ds(g * _L, _L)
        cv[sl] = (wv[sl]
                  * plsc.load_gather(ns_v, [sv[sl]])
                  * plsc.load_gather(nd_v, [dv[sl]]))

    pltpu.sync_copy(cv, cf_hbm.at[pl.ds(base, _EPW)])


@functools.cache
def _coef_kernel():
    mesh = plsc.VectorSubcoreMesh(core_axis_name="c", subcore_axis_name="s")
    return pl.kernel(
        _coef_body,
        out_type=jax.ShapeDtypeStruct((_EPAD,), jnp.float32),
        mesh=mesh,
        scratch_types=[
            pltpu.VMEM((_EPW,), jnp.int32),
            pltpu.VMEM((_EPW,), jnp.int32),
            pltpu.VMEM((_EPW,), jnp.float32),
            pltpu.VMEM((_EPW,), jnp.float32),
            pltpu.VMEM((_HNP,), jnp.float32),
            pltpu.VMEM((_HNP,), jnp.float32),
        ],
        compiler_params=_sc_compiler_params(),
    )


def _prep_body(hist_ref, w_ref, mask_ref, sloop_ref,
               ns_ref, nd_ref, c_ref, we_ref):
    deg = jnp.sum(hist_ref[...], axis=0) + 1.0
    ns = lax.rsqrt(deg[0])
    nd = lax.rsqrt(deg[1])
    ns_ref[...] = ns
    nd_ref[...] = nd
    c_ref[...] = sloop_ref[...] * ns * nd
    we_ref[...] = w_ref[...] * mask_ref[...]


def _prep_call(hist, w2, mask2, sloop):
    return pl.pallas_call(
        _prep_body,
        out_shape=(
            jax.ShapeDtypeStruct((_HNP,), jnp.float32),
            jax.ShapeDtypeStruct((_HNP,), jnp.float32),
            jax.ShapeDtypeStruct((_HNP,), jnp.float32),
            jax.ShapeDtypeStruct(w2.shape, jnp.float32),
        ),
    )(hist, w2, mask2, sloop)


def _comb1_body(acc_ref, h_ref, c_ref, out_ref):
    h = h_ref[...]
    out_ref[...] = acc_ref[0] + acc_ref[1] + c_ref[...] * h + h


def _comb1_call(acc, h, c2):
    return pl.pallas_call(
        _comb1_body,
        out_shape=jax.ShapeDtypeStruct((N, D), jnp.float32),
    )(acc, h, c2)


def _comb2_body(acc_ref, h1_ref, x_ref, c_ref, out_ref):
    h1 = h1_ref[...]
    h2 = acc_ref[0] + acc_ref[1] + c_ref[...] * h1 + h1
    out_ref[...] = (x_ref[...] + h1 + h2) * (1.0 / 3.0)


def _comb2_call(acc, h1, x, c2):
    return pl.pallas_call(
        _comb2_body,
        out_shape=jax.ShapeDtypeStruct((N, D), jnp.float32),
    )(acc, h1, x, c2)


def kernel(x, edge_index, w):
    src = edge_index[0]
    dst = edge_index[1]
    pad = _EPAD - E

    padN = jnp.full((pad,), N, jnp.int32)
    hist = _deg_kernel()(jnp.concatenate([src, padN]),
                         jnp.concatenate([dst, padN]))

    w2 = w.reshape(E // D, D)
    mask2 = jnp.asarray(_MASK_E.reshape(E // D, D))
    sloop = jnp.asarray(np.pad(_S_LOOP, (0, _HNP - N)))
    ns, nd, cvec, we = _prep_call(hist, w2, mask2, sloop)

    pad0 = jnp.zeros((pad,), jnp.int32)
    src1 = jnp.concatenate([src, pad0])
    dst1 = jnp.concatenate([dst, pad0])
    we1 = jnp.concatenate([we.reshape(E), jnp.zeros((pad,), jnp.float32)])
    cf1 = _coef_kernel()(src1, dst1, we1, ns, nd)

    src2 = src1.reshape(_EPAD // _C, _C)
    dst2 = dst1.reshape(_EPAD // _C, _C)
    cf2 = cf1.reshape(_EPAD // _C, _C)
    zeros = jnp.zeros((N, D), jnp.float32)
    c2 = cvec[:N].reshape(N, 1)

    conv = _conv_kernel()
    acc1 = conv(x, zeros, src2, dst2, cf2)
    h1 = _comb1_call(acc1, x, c2)
    acc2 = conv(h1, zeros, src2, dst2, cf2)
    return _comb2_call(acc2, h1, x, c2)


# speedup vs baseline: 1.6160x; 1.6160x over previous
import dataclasses
import functools

import numpy as np
import jax
import jax.numpy as jnp
from jax import lax
from jax.experimental import pallas as pl
from jax.experimental.pallas import tpu as pltpu
from jax.experimental.pallas import tpu_sc as plsc

N = 10000
E = 320000
D = 128
NUM_E = E + N

_NC = 2
_NS = 16
_L = 16
_NW = _NC * _NS
_C = 128
_NCH = 80
_EPW = _NCH * _C
_EPAD = _NW * _EPW
_HNP = 10112
_WSTEP = 624
_WR = 640


def _threefry2x32(k1, k2, x0, x1):
    rot_a = (13, 15, 26, 6)
    rot_b = (17, 29, 16, 24)
    u32 = np.uint32
    ks = (k1, k2, u32(k1 ^ k2 ^ u32(0x1BD11BDA)))
    x0 = (x0 + ks[0]).astype(u32)
    x1 = (x1 + ks[1]).astype(u32)

    def rounds(x0, x1, rots):
        for r in rots:
            x0 = (x0 + x1).astype(u32)
            x1 = ((x1 << u32(r)) | (x1 >> u32(32 - r))).astype(u32)
            x1 = x0 ^ x1
        return x0, x1

    for i, rots in enumerate((rot_a, rot_b, rot_a, rot_b, rot_a)):
        x0, x1 = rounds(x0, x1, rots)
        x0 = (x0 + ks[(i + 1) % 3]).astype(u32)
        x1 = (x1 + ks[(i + 2) % 3] + u32(i + 1)).astype(u32)
    return x0, x1


def _np_random_bits(key, n):
    b1, b2 = _threefry2x32(key[0], key[1],
                           np.zeros(n, np.uint32), np.arange(n, dtype=np.uint32))
    return b1 ^ b2


def _drop_mask() -> np.ndarray:
    u32 = np.uint32
    keep_prob = 1.0 - 0.3
    drop_size = int((1.0 - keep_prob) * NUM_E)
    key = (u32(0), u32(42))
    s1, s2 = _threefry2x32(key[0], key[1],
                           np.zeros(2, np.uint32), np.arange(2, dtype=np.uint32))
    k1 = (s1[0], s2[0])
    k2 = (s1[1], s2[1])
    higher = _np_random_bits(k1, drop_size)
    lower = _np_random_bits(k2, drop_size)
    span = u32(NUM_E)
    multiplier = u32((int(2 ** 16 % int(span)) ** 2 % 2 ** 32) % int(span))
    offset = ((higher % span) * multiplier + lower % span).astype(u32) % span
    drop_idx = offset.astype(np.int32)
    m = np.ones((NUM_E,), np.float32)
    m[drop_idx] = 0.0
    return m


_MASK = _drop_mask()
_MASK_E = _MASK[:E]
_S_LOOP = _MASK[E:]


def _deg_body(src_hbm, dst_hbm, out_hbm, idx_v, hs_v, hd_v):
    cid = lax.axis_index("c")
    sid = lax.axis_index("s")
    wid = sid * _NC + cid

    zeros = jnp.zeros((_L,), jnp.float32)

    @pl.loop(0, _HNP // _L)
    def _zero(r):
        hs_v[pl.ds(r * _L, _L)] = zeros
        hd_v[pl.ds(r * _L, _L)] = zeros

    ones = jnp.ones((_L,), jnp.float32)
    for t in range(2):
        src = (src_hbm, dst_hbm)[t]
        hist = (hs_v, hd_v)[t]
        pltpu.sync_copy(src.at[pl.ds(wid * _EPW, _EPW)], idx_v)

        @pl.loop(0, _EPW // _L)
        def _acc(g):
            iv = idx_v[pl.ds(g * _L, _L)]
            plsc.addupdate_scatter(hist, [iv], ones)

    pltpu.sync_copy(hs_v, out_hbm.at[wid, 0])
    pltpu.sync_copy(hd_v, out_hbm.at[wid, 1])


def _sc_compiler_params():
    cp = pltpu.CompilerParams()
    if "needs_layout_passes" in pltpu.CompilerParams.__dataclass_fields__:
        cp = dataclasses.replace(cp, needs_layout_passes=False)
    return cp


@functools.cache
def _deg_kernel():
    mesh = plsc.VectorSubcoreMesh(core_axis_name="c", subcore_axis_name="s")
    return pl.kernel(
        _deg_body,
        out_type=jax.ShapeDtypeStruct((_NW, 2, _HNP), jnp.float32),
        mesh=mesh,
        scratch_types=[
            pltpu.VMEM((_EPW,), jnp.int32),
            pltpu.VMEM((_HNP,), jnp.float32),
            pltpu.VMEM((_HNP,), jnp.float32),
        ],
        compiler_params=_sc_compiler_params(),
    )


_BLK = 16
_NCH0 = 144
_NCH1 = 2 * _NCH - _NCH0


def _scale_rows(rows, cfv, bc):
    @pl.loop(0, _C // _L)
    def _group(g):
        cv = cfv[bc, pl.ds(g * _L, _L)]
        for i in range(_L):
            wb = jnp.broadcast_to(cv[i], (_L,))
            r = g * _L + i
            for j in range(D // _L):
                fs = pl.ds(j * _L, _L)
                rows[r, fs] = rows[r, fs] * wb


def _conv_body(h_hbm, src_hbm, dst_hbm, cf_hbm, out_hbm,
               sidx, didx, cfv, rows0, rows1, sem0, sem1, acc_sh):
    cid = lax.axis_index("c")
    sid = lax.axis_index("s")

    zv = jnp.zeros((_L,), jnp.float32)

    @pl.loop(0, _C)
    def _zrow(r):
        for j in range(D // _L):
            rows0[r, pl.ds(j * _L, _L)] = zv

    row0 = sid * _WSTEP
    for k in range(_WR // _C):
        pltpu.sync_copy(rows0, acc_sh.at[pl.ds(row0 + k * _C, _C), :])
    plsc.subcore_barrier()

    cbase = jnp.where(cid == 0, sid * _NCH0, _NS * _NCH0 + sid * _NCH1)
    nblk = jnp.where(cid == 0, _NCH0 // _BLK, _NCH1 // _BLK)

    @pl.loop(0, nblk)
    def _block(b):
        bb = cbase + b * _BLK
        pltpu.sync_copy(src_hbm.at[pl.ds(bb, _BLK), :], sidx)
        pltpu.sync_copy(dst_hbm.at[pl.ds(bb, _BLK), :], didx)
        pltpu.sync_copy(cf_hbm.at[pl.ds(bb, _BLK), :], cfv)

        pltpu.async_copy(h_hbm.at[sidx.at[0]], rows0, sem0)

        @pl.loop(0, _BLK // 2)
        def _pair(p):
            c0 = 2 * p
            c1 = 2 * p + 1
            pltpu.make_async_copy(h_hbm.at[sidx.at[c0]], rows0, sem0).wait()
            pltpu.async_copy(h_hbm.at[sidx.at[c1]], rows1, sem1)
            _scale_rows(rows0, cfv, c0)
            pltpu.sync_copy(rows0, acc_sh.at[didx.at[c0]], add=True)
            pltpu.make_async_copy(h_hbm.at[sidx.at[c1]], rows1, sem1).wait()

            @pl.when(p < _BLK // 2 - 1)
            def _prefetch():
                pltpu.async_copy(h_hbm.at[sidx.at[c1 + 1]], rows0, sem0)

            _scale_rows(rows1, cfv, c1)
            pltpu.sync_copy(rows1, acc_sh.at[didx.at[c1]], add=True)

    plsc.subcore_barrier()
    pltpu.sync_copy(acc_sh.at[pl.ds(row0, _WR), :],
                    out_hbm.at[cid, pl.ds(row0, _WR), :])


@functools.cache
def _conv_kernel():
    mesh = plsc.VectorSubcoreMesh(core_axis_name="c", subcore_axis_name="s")
    return pl.kernel(
        _conv_body,
        out_type=jax.ShapeDtypeStruct((_NC, N, D), jnp.float32),
        mesh=mesh,
        scratch_types=[
            pltpu.VMEM((_BLK, _C), jnp.int32),
            pltpu.VMEM((_BLK, _C), jnp.int32),
            pltpu.VMEM((_BLK, _C), jnp.float32),
            pltpu.VMEM((_C, D), jnp.float32),
            pltpu.VMEM((_C, D), jnp.float32),
            pltpu.SemaphoreType.DMA,
            pltpu.SemaphoreType.DMA,
            pltpu.VMEM_SHARED((N, D), jnp.float32),
        ],
        compiler_params=_sc_compiler_params(),
    )


def _coef_body(src_hbm, dst_hbm, we_hbm, ns_hbm, nd_hbm, cf_hbm,
               sv, dv, wv, cv, ns_v, nd_v):
    cid = lax.axis_index("c")
    sid = lax.axis_index("s")
    wid = sid * _NC + cid
    base = wid * _EPW

    pltpu.sync_copy(ns_hbm, ns_v)
    pltpu.sync_copy(nd_hbm, nd_v)
    pltpu.sync_copy(src_hbm.at[pl.ds(base, _EPW)], sv)
    pltpu.sync_copy(dst_hbm.at[pl.ds(base, _EPW)], dv)
    pltpu.sync_copy(we_hbm.at[pl.ds(base, _EPW)], wv)

    @pl.loop(0, _EPW // _L)
    def _group(g):
        sl = pl.ds(g * _L, _L)
        cv[sl] = (wv[sl]
                  * plsc.load_gather(ns_v, [sv[sl]])
                  * plsc.load_gather(nd_v, [dv[sl]]))

    pltpu.sync_copy(cv, cf_hbm.at[pl.ds(base, _EPW)])


@functools.cache
def _coef_kernel():
    mesh = plsc.VectorSubcoreMesh(core_axis_name="c", subcore_axis_name="s")
    return pl.kernel(
        _coef_body,
        out_type=jax.ShapeDtypeStruct((_EPAD,), jnp.float32),
        mesh=mesh,
        scratch_types=[
            pltpu.VMEM((_EPW,), jnp.int32),
            pltpu.VMEM((_EPW,), jnp.int32),
            pltpu.VMEM((_EPW,), jnp.float32),
            pltpu.VMEM((_EPW,), jnp.float32),
            pltpu.VMEM((_HNP,), jnp.float32),
            pltpu.VMEM((_HNP,), jnp.float32),
        ],
        compiler_params=_sc_compiler_params(),
    )


def _prep_body(hist_ref, w_ref, mask_ref, sloop_ref,
               ns_ref, nd_ref, c_ref, we_ref):
    deg = jnp.sum(hist_ref[...], axis=0) + 1.0
    ns = lax.rsqrt(deg[0])
    nd = lax.rsqrt(deg[1])
    ns_ref[...] = ns
    nd_ref[...] = nd
    c_ref[...] = sloop_ref[...] * ns * nd
    we_ref[...] = w_ref[...] * mask_ref[...]


def _prep_call(hist, w2, mask2, sloop):
    return pl.pallas_call(
        _prep_body,
        out_shape=(
            jax.ShapeDtypeStruct((_HNP,), jnp.float32),
            jax.ShapeDtypeStruct((_HNP,), jnp.float32),
            jax.ShapeDtypeStruct((_HNP,), jnp.float32),
            jax.ShapeDtypeStruct(w2.shape, jnp.float32),
        ),
    )(hist, w2, mask2, sloop)


def _comb1_body(acc_ref, h_ref, c_ref, out_ref):
    h = h_ref[...]
    out_ref[...] = acc_ref[0] + acc_ref[1] + c_ref[...] * h + h


def _comb1_call(acc, h, c2):
    return pl.pallas_call(
        _comb1_body,
        out_shape=jax.ShapeDtypeStruct((N, D), jnp.float32),
    )(acc, h, c2)


def _comb2_body(acc_ref, h1_ref, x_ref, c_ref, out_ref):
    h1 = h1_ref[...]
    h2 = acc_ref[0] + acc_ref[1] + c_ref[...] * h1 + h1
    out_ref[...] = (x_ref[...] + h1 + h2) * (1.0 / 3.0)


def _comb2_call(acc, h1, x, c2):
    return pl.pallas_call(
        _comb2_body,
        out_shape=jax.ShapeDtypeStruct((N, D), jnp.float32),
    )(acc, h1, x, c2)


def kernel(x, edge_index, w):
    src = edge_index[0]
    dst = edge_index[1]
    pad = _EPAD - E

    padN = jnp.full((pad,), N, jnp.int32)
    hist = _deg_kernel()(jnp.concatenate([src, padN]),
                         jnp.concatenate([dst, padN]))

    w2 = w.reshape(E // D, D)
    mask2 = jnp.asarray(_MASK_E.reshape(E // D, D))
    sloop = jnp.asarray(np.pad(_S_LOOP, (0, _HNP - N)))
    ns, nd, cvec, we = _prep_call(hist, w2, mask2, sloop)

    pad0 = jnp.zeros((pad,), jnp.int32)
    src1 = jnp.concatenate([src, pad0])
    dst1 = jnp.concatenate([dst, pad0])
    we1 = jnp.concatenate([we.reshape(E), jnp.zeros((pad,), jnp.float32)])
    cf1 = _coef_kernel()(src1, dst1, we1, ns, nd)

    src2 = src1.reshape(_EPAD // _C, _C)
    dst2 = dst1.reshape(_EPAD // _C, _C)
    cf2 = cf1.reshape(_EPAD // _C, _C)
    c2 = cvec[:N].reshape(N, 1)

    conv = _conv_kernel()
    acc1 = conv(x, src2, dst2, cf2)
    h1 = _comb1_call(acc1, x, c2)
    acc2 = conv(h1, src2, dst2, cf2)
    return _comb2_call(acc2, h1, x, c2)
